# baseline (speedup 1.0000x reference)
import functools

import jax
import jax.numpy as jnp
from jax import lax
from jax.experimental import pallas as pl
from jax.experimental.pallas import tpu as pltpu
from jax.experimental.pallas import tpu_sc as plsc


def _make_gather(num_rows: int, dim: int, n_workers: int, n_chunks: int,
                 chunk: int, num_cores: int):
    mesh = plsc.VectorSubcoreMesh(core_axis_name="c", subcore_axis_name="s")

    @functools.partial(
        pl.kernel,
        mesh=mesh,
        out_type=jax.ShapeDtypeStruct((n_workers * n_chunks * chunk, dim),
                                      jnp.float32),
        scratch_types=[
            pltpu.VMEM((n_chunks, chunk), jnp.int32),
            pltpu.VMEM((chunk, dim), jnp.float32),
            pltpu.SemaphoreType.DMA,
        ],
        compiler_params=pltpu.CompilerParams(use_tc_tiling_on_sc=False),
    )
    def gather_kernel(idx_hbm, table_hbm, out_hbm, idx_v, rows_v, gsem):
        wid = lax.axis_index("s") * num_cores + lax.axis_index("c")
        pltpu.sync_copy(idx_hbm.at[wid], idx_v)
        base = wid * (n_chunks * chunk)
        for j in range(n_chunks):
            pltpu.async_copy(table_hbm.at[idx_v.at[j]], rows_v, gsem).wait()
            pltpu.sync_copy(rows_v, out_hbm.at[pl.ds(base + j * chunk, chunk)])

    return gather_kernel


def kernel(inputs, table):
    batch, seq = inputs.shape
    vocab, dim = table.shape
    total = batch * seq

    info = plsc.get_sparse_core_info()
    n_workers = info.num_cores * info.num_subcores
    per_worker = total // n_workers
    chunk = 128
    n_chunks = per_worker // chunk

    idx = inputs.astype(jnp.int32).reshape(n_workers, n_chunks, chunk)
    fn = _make_gather(vocab, dim, n_workers, n_chunks, chunk, info.num_cores)
    out = fn(idx, table)
    return out.reshape(batch, seq, dim)


# speedup vs baseline: 1.8126x; 1.8126x over previous
import functools

import jax
import jax.numpy as jnp
from jax import lax
from jax.experimental import pallas as pl
from jax.experimental.pallas import tpu as pltpu
from jax.experimental.pallas import tpu_sc as plsc


def _make_gather(n_workers: int, n_chunks: int, chunk: int, num_cores: int):
    mesh = plsc.VectorSubcoreMesh(core_axis_name="c", subcore_axis_name="s")

    @functools.partial(
        pl.kernel,
        mesh=mesh,
        out_type=jax.ShapeDtypeStruct((n_workers * n_chunks * chunk, 384),
                                      jnp.float32),
        scratch_types=[
            pltpu.VMEM((n_chunks, chunk), jnp.int32),
            pltpu.VMEM((chunk, 384), jnp.float32),
            pltpu.SemaphoreType.DMA,
        ],
    )
    def gather_kernel(idx_hbm, table_hbm, tail_hbm, out_hbm,
                      idx_v, row_v, gsem):
        wid = lax.axis_index("s") * num_cores + lax.axis_index("c")
        pltpu.sync_copy(idx_hbm.at[wid], idx_v)
        base = wid * (n_chunks * chunk)
        for j in range(n_chunks):
            c0 = pltpu.async_copy(table_hbm.at[idx_v.at[j], pl.ds(0, 128)],
                                  row_v.at[:, pl.ds(0, 128)], gsem)
            c1 = pltpu.async_copy(table_hbm.at[idx_v.at[j], pl.ds(128, 128)],
                                  row_v.at[:, pl.ds(128, 128)], gsem)
            c2 = pltpu.async_copy(tail_hbm.at[idx_v.at[j]],
                                  row_v.at[:, pl.ds(256, 128)], gsem)
            c0.wait()
            c1.wait()
            c2.wait()
            pltpu.sync_copy(row_v, out_hbm.at[pl.ds(base + j * chunk, chunk)])

    return gather_kernel


def _compact_body(dim, wide_ref, out_ref):
    out_ref[...] = wide_ref[:, :dim]


def _compact(wide, dim, block_rows=512):
    total = wide.shape[0]
    body = functools.partial(_compact_body, dim)
    return pl.pallas_call(
        body,
        grid=(total // block_rows,),
        in_specs=[pl.BlockSpec((block_rows, 384), lambda i: (i, 0))],
        out_specs=pl.BlockSpec((block_rows, dim), lambda i: (i, 0)),
        out_shape=jax.ShapeDtypeStruct((total, dim), jnp.float32),
    )(wide)


def kernel(inputs, table):
    batch, seq = inputs.shape
    vocab, dim = table.shape
    total = batch * seq

    info = plsc.get_sparse_core_info()
    n_workers = info.num_cores * info.num_subcores
    per_worker = total // n_workers
    chunk = 128
    n_chunks = per_worker // chunk

    idx = inputs.astype(jnp.int32).reshape(n_workers, n_chunks, chunk)
    tail = jnp.pad(table[:, 256:], ((0, 0), (0, 384 - dim)))
    fn = _make_gather(n_workers, n_chunks, chunk, info.num_cores)
    wide = fn(idx, table, tail)
    out = _compact(wide, dim)
    return out.reshape(batch, seq, dim)


# speedup vs baseline: 4.0820x; 2.2520x over previous
import functools

import jax
import jax.numpy as jnp
from jax import lax
from jax.experimental import pallas as pl
from jax.experimental.pallas import tpu as pltpu
from jax.experimental.pallas import tpu_sc as plsc


def _transpose_pad_body(dim, t_ref, out_ref):
    out_ref[:, :dim] = t_ref[...].T
    out_ref[:, dim:] = jnp.zeros_like(out_ref[:, dim:])


def _transpose_pad(table_t, block_rows=2048):
    dim, vocab = table_t.shape
    grid = pl.cdiv(vocab, block_rows)
    return pl.pallas_call(
        functools.partial(_transpose_pad_body, dim),
        grid=(grid,),
        in_specs=[pl.BlockSpec((dim, block_rows), lambda i: (0, i))],
        out_specs=pl.BlockSpec((block_rows, 384), lambda i: (i, 0)),
        out_shape=jax.ShapeDtypeStruct((vocab, 384), jnp.float32),
    )(table_t)


def _make_gather(n_workers: int, n_chunks: int, chunk: int, num_cores: int):
    mesh = plsc.VectorSubcoreMesh(core_axis_name="c", subcore_axis_name="s")

    @functools.partial(
        pl.kernel,
        mesh=mesh,
        out_type=jax.ShapeDtypeStruct((n_workers * n_chunks * chunk, 384),
                                      jnp.float32),
        scratch_types=[
            pltpu.VMEM((n_chunks, chunk), jnp.int32),
            pltpu.VMEM((chunk, 384), jnp.float32),
            pltpu.SemaphoreType.DMA,
        ],
    )
    def gather_kernel(idx_hbm, table_hbm, out_hbm, idx_v, row_v, gsem):
        wid = lax.axis_index("s") * num_cores + lax.axis_index("c")
        pltpu.sync_copy(idx_hbm.at[wid], idx_v)
        base = wid * (n_chunks * chunk)
        for j in range(n_chunks):
            pltpu.async_copy(table_hbm.at[idx_v.at[j]], row_v, gsem).wait()
            pltpu.sync_copy(row_v, out_hbm.at[pl.ds(base + j * chunk, chunk)])

    return gather_kernel


def _transpose_merge_body(dim, wide_ref, out_ref):
    out_ref[0, ...] = wide_ref[:, :dim].T


def _transpose_merge(wide, seq, batch, dim, block_cols=2048):
    grid = (seq, batch // block_cols)
    return pl.pallas_call(
        functools.partial(_transpose_merge_body, dim),
        grid=grid,
        in_specs=[pl.BlockSpec((block_cols, 384),
                               lambda s, b: (s * (batch // block_cols) + b, 0))],
        out_specs=pl.BlockSpec((1, dim, block_cols), lambda s, b: (s, 0, b)),
        out_shape=jax.ShapeDtypeStruct((seq, dim, batch), jnp.float32),
    )(wide)


def kernel(inputs, table):
    batch, seq = inputs.shape
    vocab, dim = table.shape
    total = batch * seq

    info = plsc.get_sparse_core_info()
    n_workers = info.num_cores * info.num_subcores
    per_worker = total // n_workers
    chunk = 128
    n_chunks = per_worker // chunk

    idx = inputs.astype(jnp.int32).T.reshape(n_workers, n_chunks, chunk)
    table_wide = _transpose_pad(table.T)
    fn = _make_gather(n_workers, n_chunks, chunk, info.num_cores)
    wide = fn(idx, table_wide)
    out_t = _transpose_merge(wide, seq, batch, dim)
    return out_t.transpose(2, 0, 1)


# speedup vs baseline: 4.2773x; 1.0478x over previous
import functools

import jax
import jax.numpy as jnp
from jax import lax
from jax.experimental import pallas as pl
from jax.experimental.pallas import tpu as pltpu
from jax.experimental.pallas import tpu_sc as plsc


def _transpose_pad_body(dim, t_ref, out_ref):
    out_ref[:, :dim] = t_ref[...].T
    out_ref[:, dim:] = jnp.zeros_like(out_ref[:, dim:])


def _transpose_pad(table_t, block_rows=2048):
    dim, vocab = table_t.shape
    grid = pl.cdiv(vocab, block_rows)
    return pl.pallas_call(
        functools.partial(_transpose_pad_body, dim),
        grid=(grid,),
        in_specs=[pl.BlockSpec((dim, block_rows), lambda i: (0, i))],
        out_specs=pl.BlockSpec((block_rows, 384), lambda i: (i, 0)),
        out_shape=jax.ShapeDtypeStruct((vocab, 384), jnp.float32),
    )(table_t)


def _make_gather(n_workers: int, n_chunks: int, chunk: int, num_cores: int):
    mesh = plsc.VectorSubcoreMesh(core_axis_name="c", subcore_axis_name="s")

    @functools.partial(
        pl.kernel,
        mesh=mesh,
        out_type=jax.ShapeDtypeStruct((n_workers * n_chunks * chunk, 384),
                                      jnp.float32),
        scratch_types=[
            pltpu.VMEM((n_chunks, chunk), jnp.int32),
            pltpu.VMEM((2, chunk, 384), jnp.float32),
            pltpu.SemaphoreType.DMA,
            pltpu.SemaphoreType.DMA,
            pltpu.SemaphoreType.DMA,
            pltpu.SemaphoreType.DMA,
        ],
    )
    def gather_kernel(idx_hbm, table_hbm, out_hbm, idx_v, row_v,
                      gsem0, gsem1, wsem0, wsem1):
        wid = lax.axis_index("s") * num_cores + lax.axis_index("c")
        pltpu.sync_copy(idx_hbm.at[wid], idx_v)
        base = wid * (n_chunks * chunk)
        gsems = (gsem0, gsem1)
        wsems = (wsem0, wsem1)
        gathers = [None, None]
        writes = [None, None]
        gathers[0] = pltpu.async_copy(
            table_hbm.at[idx_v.at[0]], row_v.at[0], gsems[0])
        for j in range(n_chunks):
            cur = j % 2
            nxt = 1 - cur
            if j + 1 < n_chunks:
                if writes[nxt] is not None:
                    writes[nxt].wait()
                gathers[nxt] = pltpu.async_copy(
                    table_hbm.at[idx_v.at[j + 1]], row_v.at[nxt], gsems[nxt])
            gathers[cur].wait()
            writes[cur] = pltpu.async_copy(
                row_v.at[cur], out_hbm.at[pl.ds(base + j * chunk, chunk)],
                wsems[cur])
        writes[0].wait()
        writes[1].wait()

    return gather_kernel


def _transpose_merge_body(dim, wide_ref, out_ref):
    out_ref[0, ...] = wide_ref[:, :dim].T


def _transpose_merge(wide, seq, batch, dim, block_cols=2048):
    grid = (seq, batch // block_cols)
    return pl.pallas_call(
        functools.partial(_transpose_merge_body, dim),
        grid=grid,
        in_specs=[pl.BlockSpec((block_cols, 384),
                               lambda s, b: (s * (batch // block_cols) + b, 0))],
        out_specs=pl.BlockSpec((1, dim, block_cols), lambda s, b: (s, 0, b)),
        out_shape=jax.ShapeDtypeStruct((seq, dim, batch), jnp.float32),
    )(wide)


def kernel(inputs, table):
    batch, seq = inputs.shape
    vocab, dim = table.shape
    total = batch * seq

    info = plsc.get_sparse_core_info()
    n_workers = info.num_cores * info.num_subcores
    per_worker = total // n_workers
    chunk = 128
    n_chunks = per_worker // chunk

    idx = inputs.astype(jnp.int32).T.reshape(n_workers, n_chunks, chunk)
    table_wide = _transpose_pad(table.T)
    fn = _make_gather(n_workers, n_chunks, chunk, info.num_cores)
    wide = fn(idx, table_wide)
    out_t = _transpose_merge(wide, seq, batch, dim)
    return out_t.transpose(2, 0, 1)
